# baseline (speedup 1.0000x reference)
import functools

import jax
import jax.numpy as jnp
from jax import lax
from jax.experimental import pallas as pl
from jax.experimental.pallas import tpu as pltpu
from jax.experimental.pallas import tpu_sc as plsc

B = 16384
D = 16
NC = 2
NS = 16
NW = NC * NS
BPW = B // NW
CW = 128
CH = BPW // CW

_mesh = plsc.VectorSubcoreMesh(core_axis_name="c", subcore_axis_name="s")
_params = pltpu.CompilerParams(
    use_tc_tiling_on_sc=False, needs_layout_passes=False)


@functools.partial(
    pl.kernel,
    out_type=jax.ShapeDtypeStruct((NW * D,), jnp.float32),
    mesh=_mesh,
    compiler_params=_params,
    scratch_types=[
        pltpu.VMEM((BPW,), jnp.int32),
        pltpu.VMEM((BPW,), jnp.int32),
        pltpu.VMEM((D, BPW), jnp.float32),
        pltpu.VMEM((D, BPW), jnp.float32),
        pltpu.VMEM((D,), jnp.float32),
        pltpu.SemaphoreType.DMA,
    ],
)
def _sc_dot(uidx_hbm, bidx_hbm, uembT_hbm, bembT_hbm, partials_hbm,
            uidx_v, bidx_v, uvals_v, bvals_v, acc_v, sem_rows):
    c = lax.axis_index("c")
    s = lax.axis_index("s")
    wid = s * NC + c
    base = wid * BPW

    pltpu.sync_copy(uidx_hbm.at[pl.ds(base, BPW)], uidx_v)
    pltpu.sync_copy(bidx_hbm.at[pl.ds(base, BPW)], bidx_v)

    for d in range(D):
        for j in range(CH):
            sl = pl.ds(j * CW, CW)
            pltpu.async_copy(
                uembT_hbm.at[d].at[uidx_v.at[sl]], uvals_v.at[d, sl],
                sem_rows)
            pltpu.async_copy(
                bembT_hbm.at[d].at[bidx_v.at[sl]], bvals_v.at[d, sl],
                sem_rows)
    pltpu.make_async_copy(
        uembT_hbm.at[pl.ds(0, D), pl.ds(0, BPW)], uvals_v, sem_rows).wait()
    pltpu.make_async_copy(
        bembT_hbm.at[pl.ds(0, D), pl.ds(0, BPW)], bvals_v, sem_rows).wait()

    def dot_group(g, acc):
        sl = pl.ds(g * 16, 16)
        for d in range(D):
            acc = acc + uvals_v[d, sl] * bvals_v[d, sl]
        return acc

    acc = lax.fori_loop(0, BPW // 16, dot_group,
                        jnp.zeros((16,), jnp.float32))
    acc_v[...] = acc
    pltpu.sync_copy(acc_v, partials_hbm.at[pl.ds(wid * D, D)])


@functools.partial(
    pl.kernel,
    out_type=jax.ShapeDtypeStruct((B,), jnp.float32),
    mesh=_mesh,
    compiler_params=_params,
    scratch_types=[
        pltpu.VMEM((BPW,), jnp.int32),
        pltpu.VMEM((BPW,), jnp.int32),
        pltpu.VMEM((BPW, 1), jnp.float32),
        pltpu.VMEM((BPW, 1), jnp.float32),
        pltpu.VMEM((NW * D,), jnp.float32),
        pltpu.VMEM((BPW,), jnp.float32),
        pltpu.SemaphoreType.DMA,
    ],
)
def _sc_bias_sigmoid(uidx_hbm, bidx_hbm, ubias_hbm, bbias_hbm, parts_hbm,
                     out_hbm, uidx_v, bidx_v, ubias_v, bbias_v, parts_v,
                     out_v, sem):
    c = lax.axis_index("c")
    s = lax.axis_index("s")
    wid = s * NC + c
    base = wid * BPW

    pltpu.sync_copy(uidx_hbm.at[pl.ds(base, BPW)], uidx_v)
    pltpu.sync_copy(bidx_hbm.at[pl.ds(base, BPW)], bidx_v)
    pltpu.sync_copy(parts_hbm, parts_v)

    descs = []
    for j in range(CH):
        sl = pl.ds(j * CW, CW)
        descs.append(pltpu.async_copy(
            ubias_hbm.at[uidx_v.at[sl]], ubias_v.at[sl], sem))
        descs.append(pltpu.async_copy(
            bbias_hbm.at[bidx_v.at[sl]], bbias_v.at[sl], sem))

    acc = jnp.zeros((16,), jnp.float32)
    for k in range(NW * D // 16):
        acc = acc + parts_v[pl.ds(k * 16, 16)]
    total = jnp.sum(acc) + jnp.zeros((16,), jnp.float32)

    for d in descs:
        d.wait()
    lane = lax.iota(jnp.int32, 16)
    zero = jnp.zeros((16,), jnp.int32)
    for g in range(BPW // 16):
        rows = lane + g * 16
        ubv = plsc.load_gather(ubias_v, [rows, zero])
        bbv = plsc.load_gather(bbias_v, [rows, zero])
        x = ubv + bbv + total
        out_v[pl.ds(g * 16, 16)] = 1.0 / (1.0 + jnp.exp(-x))
    pltpu.sync_copy(out_v, out_hbm.at[pl.ds(base, BPW)])


def kernel(inputs, user_embedding, user_bias, book_embedding, book_bias):
    uidx = inputs[:, 0].astype(jnp.int32)
    bidx = inputs[:, 1].astype(jnp.int32)
    partials = _sc_dot(uidx, bidx, user_embedding.T, book_embedding.T)
    out = _sc_bias_sigmoid(uidx, bidx, user_bias, book_bias, partials)
    return out.reshape(B, 1)


# speedup vs baseline: 1.6933x; 1.6933x over previous
import functools

import jax
import jax.numpy as jnp
from jax import lax
from jax.experimental import pallas as pl
from jax.experimental.pallas import tpu as pltpu
from jax.experimental.pallas import tpu_sc as plsc

B = 16384
D = 16
NC = 2
NS = 16
NW = NC * NS
BPW = B // NW
CW = 128
CH = BPW // CW

_mesh = plsc.VectorSubcoreMesh(core_axis_name="c", subcore_axis_name="s")
_params = pltpu.CompilerParams(
    use_tc_tiling_on_sc=False, needs_layout_passes=False)


@functools.partial(
    pl.kernel,
    out_type=jax.ShapeDtypeStruct((NW * D,), jnp.float32),
    mesh=_mesh,
    compiler_params=_params,
    scratch_types=[
        pltpu.VMEM((BPW,), jnp.int32),
        pltpu.VMEM((BPW,), jnp.int32),
        pltpu.VMEM((BPW, D), jnp.float32),
        pltpu.VMEM((BPW, D), jnp.float32),
        pltpu.VMEM((D,), jnp.float32),
        pltpu.SemaphoreType.DMA,
    ],
)
def _sc_dot(uidx_hbm, bidx_hbm, uemb_hbm, bemb_hbm, partials_hbm,
            uidx_v, bidx_v, urows_v, brows_v, acc_v, sem_rows):
    c = lax.axis_index("c")
    s = lax.axis_index("s")
    wid = s * NC + c
    base = wid * BPW

    pltpu.sync_copy(uidx_hbm.at[pl.ds(base, BPW)], uidx_v)
    pltpu.sync_copy(bidx_hbm.at[pl.ds(base, BPW)], bidx_v)

    descs = []
    for j in range(CH):
        sl = pl.ds(j * CW, CW)
        descs.append(pltpu.async_copy(
            uemb_hbm.at[uidx_v.at[sl]], urows_v.at[sl], sem_rows))
        descs.append(pltpu.async_copy(
            bemb_hbm.at[bidx_v.at[sl]], brows_v.at[sl], sem_rows))
    for d in descs:
        d.wait()

    def dot_body(i, acc):
        return acc + urows_v[i] * brows_v[i]

    acc = lax.fori_loop(0, BPW, dot_body, jnp.zeros((D,), jnp.float32))
    acc_v[...] = acc
    pltpu.sync_copy(acc_v, partials_hbm.at[pl.ds(wid * D, D)])


@functools.partial(
    pl.kernel,
    out_type=jax.ShapeDtypeStruct((B,), jnp.float32),
    mesh=_mesh,
    compiler_params=_params,
    scratch_types=[
        pltpu.VMEM((BPW,), jnp.int32),
        pltpu.VMEM((BPW,), jnp.int32),
        pltpu.VMEM((BPW, 1), jnp.float32),
        pltpu.VMEM((BPW, 1), jnp.float32),
        pltpu.VMEM((NW * D,), jnp.float32),
        pltpu.VMEM((BPW,), jnp.float32),
        pltpu.SemaphoreType.DMA,
    ],
)
def _sc_bias_sigmoid(uidx_hbm, bidx_hbm, ubias_hbm, bbias_hbm, parts_hbm,
                     out_hbm, uidx_v, bidx_v, ubias_v, bbias_v, parts_v,
                     out_v, sem):
    c = lax.axis_index("c")
    s = lax.axis_index("s")
    wid = s * NC + c
    base = wid * BPW

    pltpu.sync_copy(uidx_hbm.at[pl.ds(base, BPW)], uidx_v)
    pltpu.sync_copy(bidx_hbm.at[pl.ds(base, BPW)], bidx_v)
    pltpu.sync_copy(parts_hbm, parts_v)

    descs = []
    for j in range(CH):
        sl = pl.ds(j * CW, CW)
        descs.append(pltpu.async_copy(
            ubias_hbm.at[uidx_v.at[sl]], ubias_v.at[sl], sem))
        descs.append(pltpu.async_copy(
            bbias_hbm.at[bidx_v.at[sl]], bbias_v.at[sl], sem))

    acc = jnp.zeros((16,), jnp.float32)
    for k in range(NW * D // 16):
        acc = acc + parts_v[pl.ds(k * 16, 16)]
    total = jnp.sum(acc) + jnp.zeros((16,), jnp.float32)

    for d in descs:
        d.wait()
    lane = lax.iota(jnp.int32, 16)
    zero = jnp.zeros((16,), jnp.int32)
    for g in range(BPW // 16):
        rows = lane + g * 16
        ubv = plsc.load_gather(ubias_v, [rows, zero])
        bbv = plsc.load_gather(bbias_v, [rows, zero])
        x = ubv + bbv + total
        out_v[pl.ds(g * 16, 16)] = 1.0 / (1.0 + jnp.exp(-x))
    pltpu.sync_copy(out_v, out_hbm.at[pl.ds(base, BPW)])


def kernel(inputs, user_embedding, user_bias, book_embedding, book_bias):
    uidx = inputs[:, 0].astype(jnp.int32)
    bidx = inputs[:, 1].astype(jnp.int32)
    partials = _sc_dot(uidx, bidx, user_embedding, book_embedding)
    out = _sc_bias_sigmoid(uidx, bidx, user_bias, book_bias, partials)
    return out.reshape(B, 1)


# speedup vs baseline: 5.3136x; 3.1380x over previous
import functools

import jax
import jax.numpy as jnp
from jax import lax
from jax.experimental import pallas as pl
from jax.experimental.pallas import tpu as pltpu
from jax.experimental.pallas import tpu_sc as plsc

B = 16384
D = 16
NC = 2
NS = 16
NW = NC * NS
BPW = B // NW
CW = 128
CH = BPW // CW

_mesh = plsc.VectorSubcoreMesh(core_axis_name="c", subcore_axis_name="s")


@functools.partial(
    pl.kernel,
    out_type=(
        jax.ShapeDtypeStruct((NW * D,), jnp.float32),
        jax.ShapeDtypeStruct((B // CW, CW), jnp.float32),
    ),
    mesh=_mesh,
    compiler_params=pltpu.CompilerParams(use_tc_tiling_on_sc=False),
    scratch_types=[
        pltpu.VMEM((CH, CW), jnp.int32),
        pltpu.VMEM((CH, CW), jnp.int32),
        pltpu.VMEM((BPW, D), jnp.float32),
        pltpu.VMEM((BPW, D), jnp.float32),
        pltpu.VMEM((CH, CW), jnp.float32),
        pltpu.VMEM((CH, CW), jnp.float32),
        pltpu.VMEM((CH, CW), jnp.float32),
        pltpu.VMEM((D,), jnp.float32),
        pltpu.SemaphoreType.DMA,
    ],
)
def _sc_gather_dot(uidx_hbm, bidx_hbm, uemb_hbm, bemb_hbm, ubias_hbm,
                   bbias_hbm, partials_hbm, bsum_hbm, uidx_v, bidx_v,
                   urows_v, brows_v, ubias_v, bbias_v, bsum_v, acc_v, sem):
    c = lax.axis_index("c")
    s = lax.axis_index("s")
    wid = s * NC + c
    rowbase = wid * CH

    pltpu.sync_copy(uidx_hbm.at[pl.ds(rowbase, CH)], uidx_v)
    pltpu.sync_copy(bidx_hbm.at[pl.ds(rowbase, CH)], bidx_v)

    descs = []
    for j in range(CH):
        descs.append(pltpu.async_copy(
            uemb_hbm.at[uidx_v.at[j]], urows_v.at[pl.ds(j * CW, CW)], sem))
        descs.append(pltpu.async_copy(
            bemb_hbm.at[bidx_v.at[j]], brows_v.at[pl.ds(j * CW, CW)], sem))
        descs.append(pltpu.async_copy(
            ubias_hbm.at[uidx_v.at[j]], ubias_v.at[j], sem))
        descs.append(pltpu.async_copy(
            bbias_hbm.at[bidx_v.at[j]], bbias_v.at[j], sem))
    for d in descs:
        d.wait()

    def body(i, acc):
        return acc + urows_v[i] * brows_v[i]

    acc = lax.fori_loop(0, BPW, body, jnp.zeros((D,), jnp.float32))
    acc_v[...] = acc
    pltpu.sync_copy(acc_v, partials_hbm.at[pl.ds(wid * D, D)])

    for j in range(CH):
        for k in range(CW // 16):
            sl = pl.ds(k * 16, 16)
            bsum_v[j, sl] = ubias_v[j, sl] + bbias_v[j, sl]
    pltpu.sync_copy(bsum_v, bsum_hbm.at[pl.ds(rowbase, CH)])


def _combine_body(p_ref, b_ref, o_ref):
    total = jnp.sum(p_ref[...])
    o_ref[...] = jax.nn.sigmoid(b_ref[...] + total)


_combine = pl.pallas_call(
    _combine_body,
    out_shape=jax.ShapeDtypeStruct((B // CW, CW), jnp.float32),
)


def kernel(inputs, user_embedding, user_bias, book_embedding, book_bias):
    uidx = inputs[:, 0].astype(jnp.int32).reshape(B // CW, CW)
    bidx = inputs[:, 1].astype(jnp.int32).reshape(B // CW, CW)
    ub = user_bias.reshape(-1)
    bb = book_bias.reshape(-1)
    partials, bsums = _sc_gather_dot(
        uidx, bidx, user_embedding, book_embedding, ub, bb)
    out = _combine(partials.reshape(NW * D // CW, CW), bsums)
    return out.reshape(B, 1)
